# baseline (speedup 1.0000x reference)
import dataclasses
import functools

import jax
import jax.numpy as jnp
from jax import lax
from jax.experimental import pallas as pl
from jax.experimental.pallas import tpu as pltpu
from jax.experimental.pallas import tpu_sc as plsc

N_SRC = 80000
N_DES = 120000
D = 512
NW = 32
LANES = 16

_MESH = plsc.VectorSubcoreMesh(
    core_axis_name="c", subcore_axis_name="s", num_cores=2, num_subcores=16
)

_CP = pltpu.CompilerParams()
if "needs_layout_passes" in pltpu.CompilerParams.__dataclass_fields__:
    _CP = dataclasses.replace(_CP, needs_layout_passes=False)

C1 = 960
NCH1 = N_DES // C1
IT1 = -(-NCH1 // NW)


@functools.partial(
    pl.kernel,
    out_type=jax.ShapeDtypeStruct((N_DES,), jnp.int32),
    mesh=_MESH,
    compiler_params=_CP,
    scratch_types=[
        pltpu.VMEM((N_SRC,), jnp.int32),
        pltpu.VMEM((C1,), jnp.int32),
        pltpu.VMEM((C1,), jnp.int32),
    ],
)
def _search_kernel(sk_hbm, dk_hbm, idx_hbm, sk_v, dk_v, res_v):
    wid = lax.axis_index("s") * 2 + lax.axis_index("c")
    pltpu.sync_copy(sk_hbm, sk_v)

    @pl.loop(0, IT1)
    def _(k):
        c = k * NW + wid

        @pl.when(c < NCH1)
        def _():
            base = c * C1
            pltpu.sync_copy(dk_hbm.at[pl.ds(base, C1)], dk_v)

            @pl.loop(0, C1 // LANES)
            def _(v):
                key = dk_v[pl.ds(v * LANES, LANES)]
                pos = jnp.zeros((LANES,), jnp.int32)
                for bit in (65536, 32768, 16384, 8192, 4096, 2048, 1024,
                            512, 256, 128, 64, 32, 16, 8, 4, 2, 1):
                    np_ = pos + bit
                    gi = jnp.minimum(np_, N_SRC) - 1
                    sv = plsc.load_gather(sk_v, [gi])
                    pred = (np_ <= N_SRC) & (sv < key)
                    pos = jnp.where(pred, np_, pos)
                pos_c = jnp.minimum(pos, N_SRC - 1)
                sv = plsc.load_gather(sk_v, [pos_c])
                res = jnp.where(sv == key, pos_c, -1)
                res_v[pl.ds(v * LANES, LANES)] = res

            pltpu.sync_copy(res_v, idx_hbm.at[pl.ds(base, C1)])


R = 96
NCH2 = N_DES // R
IT2 = -(-NCH2 // NW)


@functools.partial(
    pl.kernel,
    out_type=jax.ShapeDtypeStruct((N_DES, D), jnp.float32),
    mesh=_MESH,
    compiler_params=_CP,
    scratch_types=[
        pltpu.VMEM((R,), jnp.int32),
        pltpu.VMEM((R,), jnp.int32),
        pltpu.VMEM((R, D), jnp.float32),
    ],
)
def _gather_kernel(src_hbm, idx_hbm, out_hbm, idx_v, gidx_v, rows_v):
    wid = lax.axis_index("s") * 2 + lax.axis_index("c")

    @pl.loop(0, IT2)
    def _(k):
        c = k * NW + wid

        @pl.when(c < NCH2)
        def _():
            base = c * R
            pltpu.sync_copy(idx_hbm.at[pl.ds(base, R)], idx_v)

            @pl.loop(0, R // LANES)
            def _(v):
                iv = idx_v[pl.ds(v * LANES, LANES)]
                gidx_v[pl.ds(v * LANES, LANES)] = jnp.maximum(iv, 0)

            pltpu.sync_copy(src_hbm.at[gidx_v], rows_v)

            zero = jnp.zeros((LANES,), jnp.float32)

            @pl.loop(0, R // LANES)
            def _(v):
                iv = idx_v[pl.ds(v * LANES, LANES)]
                for lane in range(LANES):
                    m = iv[lane]
                    r = v * LANES + lane

                    @pl.when(m < 0)
                    def _():
                        @pl.loop(0, D // LANES)
                        def _(j):
                            rows_v[r, pl.ds(j * LANES, LANES)] = zero

            pltpu.sync_copy(rows_v, out_hbm.at[pl.ds(base, R)])


def kernel(src_data, src_keys, des_keys):
    idx = _search_kernel(src_keys, des_keys)
    return _gather_kernel(src_data, idx)


# speedup vs baseline: 5.4876x; 5.4876x over previous
import dataclasses
import functools

import jax
import jax.numpy as jnp
from jax import lax
from jax.experimental import pallas as pl
from jax.experimental.pallas import tpu as pltpu
from jax.experimental.pallas import tpu_sc as plsc

N_SRC = 80000
N_DES = 120000
D = 512
NW = 32
LANES = 16

_MESH = plsc.VectorSubcoreMesh(
    core_axis_name="c", subcore_axis_name="s", num_cores=2, num_subcores=16
)

_CP = pltpu.CompilerParams()
if "needs_layout_passes" in pltpu.CompilerParams.__dataclass_fields__:
    _CP = dataclasses.replace(_CP, needs_layout_passes=False)

C1 = 960
NCH1 = N_DES // C1
IT1 = -(-NCH1 // NW)


@functools.partial(
    pl.kernel,
    out_type=jax.ShapeDtypeStruct((N_DES,), jnp.int32),
    mesh=_MESH,
    compiler_params=_CP,
    scratch_types=[
        pltpu.VMEM((N_SRC,), jnp.int32),
        pltpu.VMEM((C1,), jnp.int32),
        pltpu.VMEM((C1,), jnp.int32),
    ],
)
def _search_kernel(sk_hbm, dk_hbm, idx_hbm, sk_v, dk_v, res_v):
    wid = lax.axis_index("s") * 2 + lax.axis_index("c")
    pltpu.sync_copy(sk_hbm, sk_v)

    @pl.loop(0, IT1)
    def _(k):
        c = k * NW + wid

        @pl.when(c < NCH1)
        def _():
            base = c * C1
            pltpu.sync_copy(dk_hbm.at[pl.ds(base, C1)], dk_v)

            @pl.loop(0, C1 // LANES)
            def _(v):
                key = dk_v[pl.ds(v * LANES, LANES)]
                pos = jnp.zeros((LANES,), jnp.int32)
                for bit in (65536, 32768, 16384, 8192, 4096, 2048, 1024,
                            512, 256, 128, 64, 32, 16, 8, 4, 2, 1):
                    np_ = pos + bit
                    gi = jnp.minimum(np_, N_SRC) - 1
                    sv = plsc.load_gather(sk_v, [gi])
                    pred = (np_ <= N_SRC) & (sv < key)
                    pos = jnp.where(pred, np_, pos)
                pos_c = jnp.minimum(pos, N_SRC - 1)
                sv = plsc.load_gather(sk_v, [pos_c])
                res = jnp.where(sv == key, pos_c, ~pos_c)
                res_v[pl.ds(v * LANES, LANES)] = res

            pltpu.sync_copy(res_v, idx_hbm.at[pl.ds(base, C1)])


R = 96
NCH2 = N_DES // R
IT2 = -(-NCH2 // NW)


@functools.partial(
    pl.kernel,
    out_type=jax.ShapeDtypeStruct((N_DES, D), jnp.float32),
    mesh=_MESH,
    compiler_params=_CP,
    scratch_types=[
        pltpu.VMEM((R,), jnp.int32),
        pltpu.VMEM((R,), jnp.int32),
        pltpu.VMEM((R, D), jnp.float32),
    ],
)
def _gather_kernel(src_hbm, idx_hbm, out_hbm, idx_v, gidx_v, rows_v):
    wid = lax.axis_index("s") * 2 + lax.axis_index("c")

    @pl.loop(0, IT2)
    def _(k):
        c = k * NW + wid

        @pl.when(c < NCH2)
        def _():
            base = c * R
            pltpu.sync_copy(idx_hbm.at[pl.ds(base, R)], idx_v)

            @pl.loop(0, R // LANES)
            def _(v):
                iv = idx_v[pl.ds(v * LANES, LANES)]
                gidx_v[pl.ds(v * LANES, LANES)] = iv ^ (iv >> 31)

            pltpu.sync_copy(src_hbm.at[gidx_v], rows_v)

            zero = jnp.zeros((LANES,), jnp.float32)

            @pl.loop(0, R // LANES)
            def _(v):
                iv = idx_v[pl.ds(v * LANES, LANES)]
                for lane in range(LANES):
                    m = iv[lane]
                    r = v * LANES + lane

                    @pl.when(m < 0)
                    def _():
                        @pl.loop(0, D // LANES)
                        def _(j):
                            rows_v[r, pl.ds(j * LANES, LANES)] = zero

            pltpu.sync_copy(rows_v, out_hbm.at[pl.ds(base, R)])


def kernel(src_data, src_keys, des_keys):
    idx = _search_kernel(src_keys, des_keys)
    return _gather_kernel(src_data, idx)


# speedup vs baseline: 13.6972x; 2.4960x over previous
import dataclasses
import functools

import jax
import jax.numpy as jnp
from jax import lax
from jax.experimental import pallas as pl
from jax.experimental.pallas import tpu as pltpu
from jax.experimental.pallas import tpu_sc as plsc

N_SRC = 80000
N_DES = 120000
D = 512
NW = 32
LANES = 16

_MESH = plsc.VectorSubcoreMesh(
    core_axis_name="c", subcore_axis_name="s", num_cores=2, num_subcores=16
)

_CP = pltpu.CompilerParams()
if "needs_layout_passes" in pltpu.CompilerParams.__dataclass_fields__:
    _CP = dataclasses.replace(_CP, needs_layout_passes=False)

C1 = 960
NCH1 = N_DES // C1
IT1 = -(-NCH1 // NW)


@functools.partial(
    pl.kernel,
    out_type=jax.ShapeDtypeStruct((N_DES,), jnp.int32),
    mesh=_MESH,
    compiler_params=_CP,
    scratch_types=[
        pltpu.VMEM((N_SRC,), jnp.int32),
        pltpu.VMEM((C1,), jnp.int32),
        pltpu.VMEM((C1,), jnp.int32),
    ],
)
def _search_kernel(sk_hbm, dk_hbm, idx_hbm, sk_v, dk_v, res_v):
    wid = lax.axis_index("s") * 2 + lax.axis_index("c")
    pltpu.sync_copy(sk_hbm, sk_v)

    @pl.loop(0, IT1)
    def _(k):
        c = k * NW + wid

        @pl.when(c < NCH1)
        def _():
            base = c * C1
            pltpu.sync_copy(dk_hbm.at[pl.ds(base, C1)], dk_v)

            @pl.loop(0, C1 // LANES)
            def _(v):
                key = dk_v[pl.ds(v * LANES, LANES)]
                pos = jnp.zeros((LANES,), jnp.int32)
                for bit in (65536, 32768, 16384, 8192, 4096, 2048, 1024,
                            512, 256, 128, 64, 32, 16, 8, 4, 2, 1):
                    np_ = pos + bit
                    gi = jnp.minimum(np_, N_SRC) - 1
                    sv = plsc.load_gather(sk_v, [gi])
                    pred = (np_ <= N_SRC) & (sv < key)
                    pos = jnp.where(pred, np_, pos)
                pos_c = jnp.minimum(pos, N_SRC - 1)
                sv = plsc.load_gather(sk_v, [pos_c])
                res = jnp.where(sv == key, pos_c, ~pos_c)
                res_v[pl.ds(v * LANES, LANES)] = res

            pltpu.sync_copy(res_v, idx_hbm.at[pl.ds(base, C1)])


R = 96
NCH2 = N_DES // R
IT2 = -(-NCH2 // NW)


def _zero_fix(idx_ref, rows_ref):
    zero = jnp.zeros((LANES,), jnp.float32)

    @pl.loop(0, R // LANES)
    def _(v):
        iv = idx_ref[pl.ds(v * LANES, LANES)]
        for lane in range(LANES):
            m = iv[lane]
            r = v * LANES + lane

            @pl.when(m < 0)
            def _():
                for j in range(D // LANES):
                    rows_ref[r, pl.ds(j * LANES, LANES)] = zero


@functools.partial(
    pl.kernel,
    out_type=jax.ShapeDtypeStruct((N_DES, D), jnp.float32),
    mesh=_MESH,
    compiler_params=_CP,
    scratch_types=[
        pltpu.VMEM((4, R), jnp.int32),
        pltpu.VMEM((2, R), jnp.int32),
        pltpu.VMEM((R, D), jnp.float32),
        pltpu.VMEM((R, D), jnp.float32),
        pltpu.SemaphoreType.DMA((4,)),
        pltpu.SemaphoreType.DMA((2,)),
        pltpu.SemaphoreType.DMA((2,)),
    ],
)
def _gather_kernel(src_hbm, idx_hbm, out_hbm, idx4, gidx2, rows0, rows1,
                   isem, gsem, wsem):
    wid = lax.axis_index("s") * 2 + lax.axis_index("c")
    rows = (rows0, rows1)

    def c_of(kk):
        return kk * NW + wid

    def idx_copy(kk, j):
        return pltpu.make_async_copy(
            idx_hbm.at[pl.ds(c_of(kk) * R, R)], idx4.at[j], isem.at[j])

    def gather_copy(b):
        return pltpu.make_async_copy(
            src_hbm.at[gidx2.at[b]], rows[b], gsem.at[b])

    def write_copy(kk, b):
        return pltpu.make_async_copy(
            rows[b], out_hbm.at[pl.ds(c_of(kk) * R, R)], wsem.at[b])

    for kk0 in range(3):
        @pl.when(c_of(kk0) < NCH2)
        def _():
            idx_copy(kk0, kk0).start()

    @pl.loop(0, IT2 // 4)
    def _(g):
        for s in range(4):
            kk = g * 4 + s
            b = s & 1

            @pl.when(c_of(kk) < NCH2)
            def _():
                idx_copy(kk, s).wait()
                gb = gidx2.at[b]

                @pl.loop(0, R // LANES)
                def _(v):
                    iv = idx4.at[s][pl.ds(v * LANES, LANES)]
                    gb[pl.ds(v * LANES, LANES)] = iv ^ (iv >> 31)

            @pl.when(jnp.logical_and(kk >= 2, c_of(kk - 2) < NCH2))
            def _():
                write_copy(kk - 2, b).wait()

            @pl.when(c_of(kk) < NCH2)
            def _():
                gather_copy(b).start()

            @pl.when(jnp.logical_and(kk >= 1, c_of(kk - 1) < NCH2))
            def _():
                bp = b ^ 1
                gather_copy(bp).wait()
                _zero_fix(idx4.at[(s - 1) % 4], rows[bp])
                write_copy(kk - 1, bp).start()

            @pl.when(c_of(kk + 3) < NCH2)
            def _():
                idx_copy(kk + 3, (s + 3) % 4).start()

    @pl.when(c_of(IT2 - 1) < NCH2)
    def _():
        bp = (IT2 - 1) & 1
        gather_copy(bp).wait()
        _zero_fix(idx4.at[(IT2 - 1) % 4], rows[bp])
        write_copy(IT2 - 1, bp).start()

    @pl.when(c_of(IT2 - 2) < NCH2)
    def _():
        write_copy(IT2 - 2, (IT2 - 2) & 1).wait()

    @pl.when(c_of(IT2 - 1) < NCH2)
    def _():
        write_copy(IT2 - 1, (IT2 - 1) & 1).wait()


def kernel(src_data, src_keys, des_keys):
    idx = _search_kernel(src_keys, des_keys)
    return _gather_kernel(src_data, idx)


# speedup vs baseline: 15.0188x; 1.0965x over previous
import dataclasses
import functools

import jax
import jax.numpy as jnp
from jax import lax
from jax.experimental import pallas as pl
from jax.experimental.pallas import tpu as pltpu
from jax.experimental.pallas import tpu_sc as plsc

N_SRC = 80000
N_DES = 120000
D = 512
NW = 32
LANES = 16

_MESH = plsc.VectorSubcoreMesh(
    core_axis_name="c", subcore_axis_name="s", num_cores=2, num_subcores=16
)

_CP = pltpu.CompilerParams()
if "needs_layout_passes" in pltpu.CompilerParams.__dataclass_fields__:
    _CP = dataclasses.replace(_CP, needs_layout_passes=False)

C1 = 960
NCH1 = N_DES // C1
IT1 = -(-NCH1 // NW)
SK_PAD = 98304
UNROLL1 = 4


@functools.partial(
    pl.kernel,
    out_type=jax.ShapeDtypeStruct((N_DES,), jnp.int32),
    mesh=_MESH,
    compiler_params=_CP,
    scratch_types=[
        pltpu.VMEM((SK_PAD,), jnp.int32),
        pltpu.VMEM((C1,), jnp.int32),
        pltpu.VMEM((C1,), jnp.int32),
    ],
)
def _search_kernel(sk_hbm, dk_hbm, idx_hbm, sk_v, dk_v, res_v):
    wid = lax.axis_index("s") * 2 + lax.axis_index("c")
    pltpu.sync_copy(sk_hbm, sk_v.at[pl.ds(0, N_SRC)])

    sentinel = jnp.full((LANES,), 2147483647, jnp.int32)

    @pl.loop(0, (SK_PAD - N_SRC) // (LANES * 4))
    def _(v):
        for u in range(4):
            sk_v[pl.ds(N_SRC + (v * 4 + u) * LANES, LANES)] = sentinel

    @pl.loop(0, IT1)
    def _(k):
        c = k * NW + wid

        @pl.when(c < NCH1)
        def _():
            base = c * C1
            pltpu.sync_copy(dk_hbm.at[pl.ds(base, C1)], dk_v)

            @pl.loop(0, C1 // (LANES * UNROLL1))
            def _(v4):
                keys, poss = [], []
                for u in range(UNROLL1):
                    keys.append(dk_v[pl.ds((v4 * UNROLL1 + u) * LANES, LANES)])
                    poss.append(jnp.zeros((LANES,), jnp.int32))
                for bit in (65536, 32768, 16384, 8192, 4096, 2048, 1024,
                            512, 256, 128, 64, 32, 16, 8, 4, 2, 1):
                    gis = [p + (bit - 1) for p in poss]
                    svs = [plsc.load_gather(sk_v, [gi]) for gi in gis]
                    poss = [jnp.where(sv < key, gi + 1, p)
                            for sv, key, gi, p in zip(svs, keys, gis, poss)]
                for u in range(UNROLL1):
                    pos = poss[u]
                    sv = plsc.load_gather(sk_v, [pos])
                    pos_c = jnp.minimum(pos, N_SRC - 1)
                    res = jnp.where(sv == keys[u], pos, ~pos_c)
                    res_v[pl.ds((v4 * UNROLL1 + u) * LANES, LANES)] = res

            pltpu.sync_copy(res_v, idx_hbm.at[pl.ds(base, C1)])


R = 96
NCH2 = N_DES // R
IT2 = -(-NCH2 // NW)


def _zero_fix(idx_ref, rows_ref):
    zero = jnp.zeros((LANES,), jnp.float32)

    @pl.loop(0, R // LANES)
    def _(v):
        iv = idx_ref[pl.ds(v * LANES, LANES)]
        for lane in range(LANES):
            m = iv[lane]
            r = v * LANES + lane

            @pl.when(m < 0)
            def _():
                for j in range(D // LANES):
                    rows_ref[r, pl.ds(j * LANES, LANES)] = zero


@functools.partial(
    pl.kernel,
    out_type=jax.ShapeDtypeStruct((N_DES, D), jnp.float32),
    mesh=_MESH,
    compiler_params=_CP,
    scratch_types=[
        pltpu.VMEM((4, R), jnp.int32),
        pltpu.VMEM((2, R), jnp.int32),
        pltpu.VMEM((R, D), jnp.float32),
        pltpu.VMEM((R, D), jnp.float32),
        pltpu.SemaphoreType.DMA((4,)),
        pltpu.SemaphoreType.DMA((2,)),
        pltpu.SemaphoreType.DMA((2,)),
    ],
)
def _gather_kernel(src_hbm, idx_hbm, out_hbm, idx4, gidx2, rows0, rows1,
                   isem, gsem, wsem):
    wid = lax.axis_index("s") * 2 + lax.axis_index("c")
    rows = (rows0, rows1)

    def c_of(kk):
        return kk * NW + wid

    def idx_copy(kk, j):
        return pltpu.make_async_copy(
            idx_hbm.at[pl.ds(c_of(kk) * R, R)], idx4.at[j], isem.at[j])

    def gather_copy(b):
        return pltpu.make_async_copy(
            src_hbm.at[gidx2.at[b]], rows[b], gsem.at[b])

    def write_copy(kk, b):
        return pltpu.make_async_copy(
            rows[b], out_hbm.at[pl.ds(c_of(kk) * R, R)], wsem.at[b])

    for kk0 in range(3):
        @pl.when(c_of(kk0) < NCH2)
        def _():
            idx_copy(kk0, kk0).start()

    @pl.loop(0, IT2 // 4)
    def _(g):
        for s in range(4):
            kk = g * 4 + s
            b = s & 1

            @pl.when(c_of(kk) < NCH2)
            def _():
                idx_copy(kk, s).wait()
                gb = gidx2.at[b]

                @pl.loop(0, R // LANES)
                def _(v):
                    iv = idx4.at[s][pl.ds(v * LANES, LANES)]
                    gb[pl.ds(v * LANES, LANES)] = iv ^ (iv >> 31)

            @pl.when(jnp.logical_and(kk >= 2, c_of(kk - 2) < NCH2))
            def _():
                write_copy(kk - 2, b).wait()

            @pl.when(c_of(kk) < NCH2)
            def _():
                gather_copy(b).start()

            @pl.when(jnp.logical_and(kk >= 1, c_of(kk - 1) < NCH2))
            def _():
                bp = b ^ 1
                gather_copy(bp).wait()
                _zero_fix(idx4.at[(s - 1) % 4], rows[bp])
                write_copy(kk - 1, bp).start()

            @pl.when(c_of(kk + 3) < NCH2)
            def _():
                idx_copy(kk + 3, (s + 3) % 4).start()

    @pl.when(c_of(IT2 - 1) < NCH2)
    def _():
        bp = (IT2 - 1) & 1
        gather_copy(bp).wait()
        _zero_fix(idx4.at[(IT2 - 1) % 4], rows[bp])
        write_copy(IT2 - 1, bp).start()

    @pl.when(c_of(IT2 - 2) < NCH2)
    def _():
        write_copy(IT2 - 2, (IT2 - 2) & 1).wait()

    @pl.when(c_of(IT2 - 1) < NCH2)
    def _():
        write_copy(IT2 - 1, (IT2 - 1) & 1).wait()


def kernel(src_data, src_keys, des_keys):
    idx = _search_kernel(src_keys, des_keys)
    return _gather_kernel(src_data, idx)


# speedup vs baseline: 15.3346x; 1.0210x over previous
import dataclasses
import functools

import jax
import jax.numpy as jnp
from jax import lax
from jax.experimental import pallas as pl
from jax.experimental.pallas import tpu as pltpu
from jax.experimental.pallas import tpu_sc as plsc

N_SRC = 80000
N_DES = 120000
D = 512
NW = 32
LANES = 16

_MESH = plsc.VectorSubcoreMesh(
    core_axis_name="c", subcore_axis_name="s", num_cores=2, num_subcores=16
)

_CP = pltpu.CompilerParams()
if "needs_layout_passes" in pltpu.CompilerParams.__dataclass_fields__:
    _CP = dataclasses.replace(_CP, needs_layout_passes=False)

C1 = 960
NCH1 = N_DES // C1
IT1 = -(-NCH1 // NW)
SK_PAD = 98304
UNROLL1 = 4


@functools.partial(
    pl.kernel,
    out_type=(
        jax.ShapeDtypeStruct((N_DES + 96,), jnp.int32),
        jax.ShapeDtypeStruct((N_DES + 96,), jnp.int32),
    ),
    mesh=_MESH,
    compiler_params=_CP,
    scratch_types=[
        pltpu.VMEM((SK_PAD,), jnp.int32),
        pltpu.VMEM((C1,), jnp.int32),
        pltpu.VMEM((C1,), jnp.int32),
        pltpu.VMEM((C1,), jnp.int32),
    ],
)
def _search_kernel(sk_hbm, dk_hbm, idx_hbm, gidx_hbm, sk_v, dk_v, res_v,
                   gres_v):
    wid = lax.axis_index("s") * 2 + lax.axis_index("c")
    pltpu.sync_copy(sk_hbm, sk_v.at[pl.ds(0, N_SRC)])

    sentinel = jnp.full((LANES,), 2147483647, jnp.int32)

    @pl.loop(0, (SK_PAD - N_SRC) // (LANES * 4))
    def _(v):
        for u in range(4):
            sk_v[pl.ds(N_SRC + (v * 4 + u) * LANES, LANES)] = sentinel

    @pl.loop(0, IT1)
    def _(k):
        c = k * NW + wid

        @pl.when(c < NCH1)
        def _():
            base = c * C1
            pltpu.sync_copy(dk_hbm.at[pl.ds(base, C1)], dk_v)

            @pl.loop(0, C1 // (LANES * UNROLL1))
            def _(v4):
                keys, poss = [], []
                for u in range(UNROLL1):
                    keys.append(dk_v[pl.ds((v4 * UNROLL1 + u) * LANES, LANES)])
                    poss.append(jnp.zeros((LANES,), jnp.int32))
                for bit in (65536, 32768, 16384, 8192, 4096, 2048, 1024,
                            512, 256, 128, 64, 32, 16, 8, 4, 2, 1):
                    gis = [p + (bit - 1) for p in poss]
                    svs = [plsc.load_gather(sk_v, [gi]) for gi in gis]
                    poss = [jnp.where(sv < key, gi + 1, p)
                            for sv, key, gi, p in zip(svs, keys, gis, poss)]
                for u in range(UNROLL1):
                    pos = poss[u]
                    sv = plsc.load_gather(sk_v, [pos])
                    pos_c = jnp.minimum(pos, N_SRC - 1)
                    res = jnp.where(sv == keys[u], pos, ~pos_c)
                    sl = pl.ds((v4 * UNROLL1 + u) * LANES, LANES)
                    res_v[sl] = res
                    gres_v[sl] = pos_c

            pltpu.sync_copy(res_v, idx_hbm.at[pl.ds(base, C1)])
            pltpu.sync_copy(gres_v, gidx_hbm.at[pl.ds(base, C1)])


R = 96
NCH2 = N_DES // R
IT2 = -(-NCH2 // NW)
NB = IT2 * R


def _zero_fix(idx_ref, kk, rows_ref):
    zero = jnp.zeros((LANES,), jnp.float32)

    @pl.loop(0, R // LANES)
    def _(v):
        iv = idx_ref[pl.ds(kk * R + v * LANES, LANES)]
        for lane in range(LANES):
            m = iv[lane]
            r = v * LANES + lane

            @pl.when(m < 0)
            def _():
                for j in range(D // LANES):
                    rows_ref[r, pl.ds(j * LANES, LANES)] = zero


@functools.partial(
    pl.kernel,
    out_type=jax.ShapeDtypeStruct((N_DES, D), jnp.float32),
    mesh=_MESH,
    compiler_params=_CP,
    scratch_types=[
        pltpu.VMEM((NB,), jnp.int32),
        pltpu.VMEM((NB,), jnp.int32),
        pltpu.VMEM((R, D), jnp.float32),
        pltpu.VMEM((R, D), jnp.float32),
        pltpu.SemaphoreType.DMA((2,)),
        pltpu.SemaphoreType.DMA((2,)),
    ],
)
def _gather_kernel(src_hbm, idx_hbm, gidx_hbm, out_hbm, idx_all, gidx_all,
                   rows0, rows1, gsem, wsem):
    wid = lax.axis_index("s") * 2 + lax.axis_index("c")
    rows = (rows0, rows1)
    base_c = wid * (IT2 - 1) + jnp.minimum(wid, NCH2 - NW * (IT2 - 1))
    nc = (IT2 - 1) + jnp.where(wid < NCH2 - NW * (IT2 - 1), 1, 0)

    def gather_copy(kk, b):
        return pltpu.make_async_copy(
            src_hbm.at[gidx_all.at[pl.ds(kk * R, R)]], rows[b], gsem.at[b])

    def write_copy(kk, b):
        return pltpu.make_async_copy(
            rows[b], out_hbm.at[pl.ds((base_c + kk) * R, R)], wsem.at[b])

    pltpu.sync_copy(idx_hbm.at[pl.ds(base_c * R, NB)], idx_all)
    pltpu.sync_copy(gidx_hbm.at[pl.ds(base_c * R, NB)], gidx_all)

    gather_copy(0, 0).start()

    @pl.loop(0, IT2 // 2)
    def _(g):
        for s in range(2):
            kk = g * 2 + s
            b = s

            @pl.when(jnp.logical_and(kk >= 1, kk - 1 < nc))
            def _():
                write_copy(kk - 1, b ^ 1).wait()

            @pl.when(kk + 1 < nc)
            def _():
                gather_copy(kk + 1, b ^ 1).start()

            @pl.when(kk < nc)
            def _():
                gather_copy(kk, b).wait()
                _zero_fix(idx_all, kk, rows[b])
                write_copy(kk, b).start()

    @pl.when(nc > IT2 - 1)
    def _():
        write_copy(IT2 - 1, (IT2 - 1) & 1).wait()


def kernel(src_data, src_keys, des_keys):
    idx, gidx = _search_kernel(src_keys, des_keys)
    return _gather_kernel(src_data, idx, gidx)
